# baseline (speedup 1.0000x reference)
import functools

import jax
import jax.numpy as jnp
from jax import lax
from jax.experimental import pallas as pl
from jax.experimental.pallas import tpu as pltpu
from jax.experimental.pallas import tpu_sc as plsc

VOCAB = 1000000
EMB = 32
HID = 128
OUT = 16
BATCH = 16384

GRP = 128 // EMB
NSLAB = VOCAB // GRP
NC = 2
NS = 16
NW = NC * NS
B_PER_W = BATCH // NW
CHUNK = 128
NCH = B_PER_W // CHUNK


def _sc_gather_body(emb_hbm, idx_hbm, out_hbm, idx_v, slabs_v, sem):
    c = lax.axis_index("c")
    s = lax.axis_index("s")
    wid = s * NC + c
    pltpu.sync_copy(idx_hbm.at[wid], idx_v)
    cps = [
        pltpu.async_copy(emb_hbm.at[idx_v.at[j]], slabs_v.at[j], sem)
        for j in range(NCH)
    ]
    for cp in cps:
        cp.wait()
    pltpu.sync_copy(slabs_v, out_hbm.at[wid])


@jax.jit
def _sc_gather(emb2, idx):
    mesh = plsc.VectorSubcoreMesh(core_axis_name="c", subcore_axis_name="s")
    k = functools.partial(
        pl.kernel,
        mesh=mesh,
        out_type=jax.ShapeDtypeStruct((NW, NCH, CHUNK, 128), jnp.float32),
        scratch_types=[
            pltpu.VMEM((NCH, CHUNK), jnp.int32),
            pltpu.VMEM((NCH, CHUNK, 128), jnp.float32),
            pltpu.SemaphoreType.DMA,
        ],
    )(_sc_gather_body)
    return k(emb2, idx)


def _mlp_body(h_ref, s_ref, w1_ref, b1_ref, w2_ref, b2_ref, w3_ref, b3_ref, o_ref):
    hw = h_ref[...]
    sel = s_ref[...]
    h = jnp.zeros((hw.shape[0], EMB), jnp.float32)
    for k in range(GRP):
        h = h + jnp.where(sel == k, hw[:, k * EMB : (k + 1) * EMB], 0.0)
    z = jnp.dot(h, w1_ref[...], preferred_element_type=jnp.float32)
    z = jnp.maximum(z + b1_ref[...], 0.0)
    z = jnp.dot(z, w2_ref[...], preferred_element_type=jnp.float32)
    z = jnp.maximum(z + b2_ref[...], 0.0)
    z = jnp.dot(z, w3_ref[...], preferred_element_type=jnp.float32)
    o_ref[...] = z + b3_ref[...]


BLK = 2048


@jax.jit
def _tc_mlp(hw, sel, W1, b1, W2, b2, W3, b3):
    grid = (BATCH // BLK,)
    full = lambda shape: pl.BlockSpec(shape, lambda i: (0, 0))
    return pl.pallas_call(
        _mlp_body,
        grid=grid,
        in_specs=[
            pl.BlockSpec((BLK, 128), lambda i: (i, 0)),
            pl.BlockSpec((BLK, 1), lambda i: (i, 0)),
            full((EMB, HID)),
            full((1, HID)),
            full((HID, HID)),
            full((1, HID)),
            full((HID, OUT)),
            full((1, OUT)),
        ],
        out_specs=pl.BlockSpec((BLK, OUT), lambda i: (i, 0)),
        out_shape=jax.ShapeDtypeStruct((BATCH, OUT), jnp.float32),
    )(hw, sel, W1, b1, W2, b2, W3, b3)


def kernel(x, emb, W1, b1, W2, b2, W3, b3):
    xi = x.astype(jnp.int32)
    emb2 = emb.reshape(NSLAB, 128)
    gidx = (xi // GRP).reshape(NW, NCH, CHUNK)
    slabs = _sc_gather(emb2, gidx).reshape(BATCH, 128)
    sel = (xi % GRP).reshape(BATCH, 1)
    return _tc_mlp(
        slabs,
        sel,
        W1,
        b1.reshape(1, HID),
        W2,
        b2.reshape(1, HID),
        W3,
        b3.reshape(1, OUT),
    )


# speedup vs baseline: 1.0343x; 1.0343x over previous
import functools

import jax
import jax.numpy as jnp
from jax import lax
from jax.experimental import pallas as pl
from jax.experimental.pallas import tpu as pltpu
from jax.experimental.pallas import tpu_sc as plsc

VOCAB = 1000000
EMB = 32
HID = 128
OUT = 16
BATCH = 16384

GRP = 8
SLAB = GRP * EMB
NSLAB = VOCAB // GRP
NC = 2
NS = 16
NW = NC * NS
B_PER_W = BATCH // NW
CHUNK = 128
NCH = B_PER_W // CHUNK
HALF = NCH // 2


def _sc_gather_body(emb2_hbm, idx_hbm, out_hbm, idx_v, slabs_v, sem):
    c = lax.axis_index("c")
    s = lax.axis_index("s")
    wid = s * NC + c
    pltpu.sync_copy(idx_hbm.at[wid], idx_v)
    for h in range(2):
        cps = [
            pltpu.async_copy(
                emb2_hbm.at[idx_v.at[h * HALF + j]], slabs_v.at[j], sem
            )
            for j in range(HALF)
        ]
        for cp in cps:
            cp.wait()
        pltpu.sync_copy(slabs_v, out_hbm.at[wid, h])


@jax.jit
def _sc_gather(emb2, idx):
    mesh = plsc.VectorSubcoreMesh(core_axis_name="c", subcore_axis_name="s")
    k = functools.partial(
        pl.kernel,
        mesh=mesh,
        out_type=jax.ShapeDtypeStruct((NW, 2, HALF, CHUNK, SLAB), jnp.float32),
        scratch_types=[
            pltpu.VMEM((NCH, CHUNK), jnp.int32),
            pltpu.VMEM((HALF, CHUNK, SLAB), jnp.float32),
            pltpu.SemaphoreType.DMA,
        ],
    )(_sc_gather_body)
    return k(emb2, idx)


RROWS = 1000


def _repack_body(in_ref, o_ref):
    hw = in_ref[...]
    o_ref[...] = jnp.concatenate([hw[:, k, :] for k in range(GRP)], axis=1)


@jax.jit
def _tc_repack(emb3):
    grid = (NSLAB // RROWS,)
    return pl.pallas_call(
        _repack_body,
        grid=grid,
        in_specs=[pl.BlockSpec((RROWS, GRP, EMB), lambda i: (i, 0, 0))],
        out_specs=pl.BlockSpec((RROWS, SLAB), lambda i: (i, 0)),
        out_shape=jax.ShapeDtypeStruct((NSLAB, SLAB), jnp.float32),
    )(emb3)


def _mlp_body(h_ref, s_ref, w1_ref, b1_ref, w2_ref, b2_ref, w3_ref, b3_ref, o_ref):
    hw = h_ref[...]
    sel = s_ref[...]
    h = jnp.zeros((hw.shape[0], EMB), jnp.float32)
    for k in range(GRP):
        h = h + jnp.where(sel == k, hw[:, k * EMB : (k + 1) * EMB], 0.0)
    z = jnp.dot(h, w1_ref[...], preferred_element_type=jnp.float32)
    z = jnp.maximum(z + b1_ref[...], 0.0)
    z = jnp.dot(z, w2_ref[...], preferred_element_type=jnp.float32)
    z = jnp.maximum(z + b2_ref[...], 0.0)
    z = jnp.dot(z, w3_ref[...], preferred_element_type=jnp.float32)
    o_ref[...] = z + b3_ref[...]


BLK = 2048


@jax.jit
def _tc_mlp(hw, sel, W1, b1, W2, b2, W3, b3):
    grid = (BATCH // BLK,)
    full = lambda shape: pl.BlockSpec(shape, lambda i: (0, 0))
    return pl.pallas_call(
        _mlp_body,
        grid=grid,
        in_specs=[
            pl.BlockSpec((BLK, SLAB), lambda i: (i, 0)),
            pl.BlockSpec((BLK, 1), lambda i: (i, 0)),
            full((EMB, HID)),
            full((1, HID)),
            full((HID, HID)),
            full((1, HID)),
            full((HID, OUT)),
            full((1, OUT)),
        ],
        out_specs=pl.BlockSpec((BLK, OUT), lambda i: (i, 0)),
        out_shape=jax.ShapeDtypeStruct((BATCH, OUT), jnp.float32),
    )(hw, sel, W1, b1, W2, b2, W3, b3)


def kernel(x, emb, W1, b1, W2, b2, W3, b3):
    xi = x.astype(jnp.int32)
    emb3 = emb.reshape(NSLAB, GRP, EMB)
    emb2 = _tc_repack(emb3)
    gidx = (xi // GRP).reshape(NW, NCH, CHUNK)
    slabs = _sc_gather(emb2, gidx).reshape(BATCH, SLAB)
    sel = (xi % GRP).reshape(BATCH, 1)
    return _tc_mlp(
        slabs,
        sel,
        W1,
        b1.reshape(1, HID),
        W2,
        b2.reshape(1, HID),
        W3,
        b3.reshape(1, OUT),
    )
